# baseline (speedup 1.0000x reference)
import functools
import math

import jax
import jax.numpy as jnp
from jax import lax
from jax.experimental import pallas as pl
from jax.experimental.pallas import tpu as pltpu
from jax.experimental.pallas import tpu_sc as plsc

D = 64
SCALE = math.sqrt(D)

_NC = 2
_NS = 16
_NW = _NC * _NS

CH = 128


@functools.partial(jax.jit, static_argnames=("n_rows",))
def _gather_scaled(xf, lut, n_rows):
    b_per_w = n_rows // _NW
    n_chunks = b_per_w // CH

    mesh = plsc.VectorSubcoreMesh(core_axis_name="c", subcore_axis_name="s")

    @functools.partial(
        pl.kernel,
        out_type=jax.ShapeDtypeStruct((n_rows, D), jnp.float32),
        mesh=mesh,
        scratch_types=[
            pltpu.VMEM((CH,), jnp.int32),
            pltpu.VMEM((CH, D), jnp.float32),
            pltpu.SemaphoreType.DMA,
        ],
        compiler_params=pltpu.CompilerParams(use_tc_tiling_on_sc=False),
    )
    def k(x_hbm, lut_hbm, out_hbm, idx_v, rows_v, sem):
        wid = lax.axis_index("s") * _NC + lax.axis_index("c")
        base = wid * b_per_w

        def chunk(i, carry):
            off = base + i * CH
            pltpu.sync_copy(x_hbm.at[pl.ds(off, CH)], idx_v)
            pltpu.async_copy(lut_hbm.at[idx_v], rows_v, sem).wait()

            def scl(t, c):
                r = t // 4
                col = (t % 4) * 16
                rows_v[r, pl.ds(col, 16)] = rows_v[r, pl.ds(col, 16)] * SCALE
                return c

            lax.fori_loop(0, CH * 4, scl, 0, unroll=4)
            pltpu.sync_copy(rows_v, out_hbm.at[pl.ds(off, CH)])
            return carry

        lax.fori_loop(0, n_chunks, chunk, 0)

    return k(xf, lut)


def kernel(x, lut):
    n_rows = x.shape[0] * x.shape[1]
    xf = x.reshape(n_rows)
    out = _gather_scaled(xf, lut, n_rows)
    return out.reshape(x.shape[0], x.shape[1], D)


# speedup vs baseline: 1.2533x; 1.2533x over previous
import functools
import math

import jax
import jax.numpy as jnp
from jax import lax
from jax.experimental import pallas as pl
from jax.experimental.pallas import tpu as pltpu
from jax.experimental.pallas import tpu_sc as plsc

D = 64
SCALE = math.sqrt(D)

_NC = 2
_NS = 16
_NW = _NC * _NS

CH = 512
NSUB = CH // 128
NB = 2


@functools.partial(jax.jit, static_argnames=("n_rows",))
def _gather_scaled(x2d, lut, n_rows):
    b_per_w = n_rows // _NW
    n_chunks = b_per_w // CH
    assert n_chunks % NB == 0

    mesh = plsc.VectorSubcoreMesh(core_axis_name="c", subcore_axis_name="s")

    @functools.partial(
        pl.kernel,
        out_type=jax.ShapeDtypeStruct((n_rows, D), jnp.float32),
        mesh=mesh,
        scratch_types=[
            pltpu.VMEM((CH,), jnp.int32),
            pltpu.VMEM((CH,), jnp.int32),
            pltpu.VMEM((CH, D), jnp.float32),
            pltpu.VMEM((CH, D), jnp.float32),
            pltpu.SemaphoreType.DMA,
            pltpu.SemaphoreType.DMA,
            pltpu.SemaphoreType.DMA,
            pltpu.SemaphoreType.DMA,
        ],
        compiler_params=pltpu.CompilerParams(use_tc_tiling_on_sc=False),
    )
    def k(x_hbm, lut_hbm, out_hbm, idx0, idx1, rows0, rows1, g0, g1, o0, o1):
        idx = (idx0, idx1)
        rows = (rows0, rows1)
        gsem = (g0, g1)
        osem = (o0, o1)

        wid = lax.axis_index("s") * _NC + lax.axis_index("c")
        base = wid * b_per_w

        def fire(i, b):
            pltpu.sync_copy(x_hbm.at[pl.ds(base + i * CH, CH)], idx[b])
            pltpu.async_copy(lut_hbm.at[idx[b]], rows[b], gsem[b])

        def drain_gather(b):
            pltpu.make_async_copy(lut_hbm.at[idx[b]], rows[b], gsem[b]).wait()

        def wait_out(b):
            pltpu.make_async_copy(
                rows[b], out_hbm.at[pl.ds(base, CH)], osem[b]
            ).wait()

        fire(0, 0)

        def body(i2, carry):
            for b in range(NB):
                i = i2 * NB + b
                nb = 1 - b

                @pl.when(i + 1 < n_chunks)
                def _():
                    @pl.when(i >= 1)
                    def _():
                        wait_out(nb)

                    fire(i + 1, nb)

                drain_gather(b)

                def scl(r, c):
                    for j in range(D // 16):
                        rows[b][r, pl.ds(j * 16, 16)] = (
                            rows[b][r, pl.ds(j * 16, 16)] * SCALE
                        )
                    return c

                lax.fori_loop(0, CH, scl, 0, unroll=4)
                pltpu.async_copy(
                    rows[b], out_hbm.at[pl.ds(base + i * CH, CH)], osem[b]
                )
            return carry

        lax.fori_loop(0, n_chunks // NB, body, 0)
        wait_out(0)
        wait_out(1)

    return k(x2d, lut)


def kernel(x, lut):
    n_rows = x.shape[0] * x.shape[1]
    xf = x.reshape(n_rows)
    out = _gather_scaled(xf, lut, n_rows)
    return out.reshape(x.shape[0], x.shape[1], D)
